# baseline (speedup 1.0000x reference)
import functools

import jax
import jax.numpy as jnp
from jax.experimental import pallas as pl
from jax.experimental.pallas import tpu as pltpu

LS = 0.1


def _tc_body(tgt_ref, lp_ref, out_ref, *, bn, bv):
    i = pl.program_id(0)
    j = pl.program_id(1)

    v_total = pl.num_programs(1) * bv
    w_miss = LS / v_total
    w_hit = 1.0 - LS

    lp = jnp.maximum(lp_ref[...], -100.0)
    tgt = tgt_ref[0, 0, :]
    valid = (tgt != 1).astype(jnp.float32)

    col = j * bv + jax.lax.broadcasted_iota(jnp.int32, (bn, bv), 1)
    w = jnp.where(col == tgt[:, None], w_hit, w_miss)
    wrow = jnp.sum(lp * w, axis=1)

    @pl.when((i == 0) & (j == 0))
    def _():
        out_ref[0] = 0.0
        out_ref[1] = 0.0

    out_ref[0] += jnp.sum(wrow * valid)

    @pl.when(j == 0)
    def _():
        out_ref[1] += jnp.sum(valid)


def kernel(log_probs, targets, triplets):
    n, v = log_probs.shape
    bn = 64
    bv = 32000
    nb = n // bn
    vb = v // bv

    tgt3 = targets.reshape(nb, 1, bn)

    sums = pl.pallas_call(
        functools.partial(_tc_body, bn=bn, bv=bv),
        grid=(nb, vb),
        in_specs=[
            pl.BlockSpec((1, 1, bn), lambda i, j: (i, 0, 0)),
            pl.BlockSpec((bn, bv), lambda i, j: (i, j)),
        ],
        out_specs=pl.BlockSpec(memory_space=pltpu.SMEM),
        out_shape=jax.ShapeDtypeStruct((2,), jnp.float32),
    )(tgt3, log_probs)

    return -sums[0] / jnp.maximum(sums[1], 1.0)


# speedup vs baseline: 1.1284x; 1.1284x over previous
import functools

import jax
import jax.numpy as jnp
from jax.experimental import pallas as pl
from jax.experimental.pallas import tpu as pltpu

LS = 0.1


def _wsum(lp, tgt, v):
    w_miss = LS / v
    w_hit = 1.0 - LS
    bn, bv = lp.shape
    lp = jnp.maximum(lp, -100.0)
    valid = (tgt != 1).astype(jnp.float32)
    col = jax.lax.broadcasted_iota(jnp.int32, (bn, bv), 1)
    w = jnp.where(col == tgt[:, None], w_hit, w_miss)
    wrow = jnp.sum(lp * w, axis=1)
    return jnp.sum(wrow * valid), jnp.sum(valid)


def _tc_body(tgt_a_ref, tgt_b_ref, lp_a_ref, lp_b_ref, out_ref, *, v):
    i = pl.program_id(0)

    sa, ca = _wsum(lp_a_ref[...], tgt_a_ref[0, 0, :], v)
    sb, cb = _wsum(lp_b_ref[...], tgt_b_ref[0, 0, :], v)

    @pl.when(i == 0)
    def _():
        out_ref[0] = 0.0
        out_ref[1] = 0.0

    out_ref[0] += sa + sb
    out_ref[1] += ca + cb


def kernel(log_probs, targets, triplets):
    n, v = log_probs.shape
    bn = 64
    nb = n // bn
    half = nb // 2

    tgt3 = targets.reshape(nb, 1, bn)

    sums = pl.pallas_call(
        functools.partial(_tc_body, v=v),
        grid=(half,),
        in_specs=[
            pl.BlockSpec((1, 1, bn), lambda i: (i, 0, 0)),
            pl.BlockSpec((1, 1, bn), lambda i: (i + half, 0, 0)),
            pl.BlockSpec((bn, v), lambda i: (i, 0)),
            pl.BlockSpec((bn, v), lambda i: (i + half, 0)),
        ],
        out_specs=pl.BlockSpec(memory_space=pltpu.SMEM),
        out_shape=jax.ShapeDtypeStruct((2,), jnp.float32),
    )(tgt3, tgt3, log_probs, log_probs)

    return -sums[0] / jnp.maximum(sums[1], 1.0)


# speedup vs baseline: 1.1291x; 1.0007x over previous
import functools

import jax
import jax.numpy as jnp
from jax.experimental import pallas as pl
from jax.experimental.pallas import tpu as pltpu

LS = 0.1


def _tc_body(tgt_ref, lp_ref, out_ref, *, bn, bv):
    i = pl.program_id(0)
    j = pl.program_id(1)

    v_total = pl.num_programs(1) * bv
    w_miss = LS / v_total
    w_hit = 1.0 - LS

    lp = jnp.maximum(lp_ref[...], -100.0)
    tgt = tgt_ref[0, 0, :]
    valid = (tgt != 1).astype(jnp.float32)

    col = j * bv + jax.lax.broadcasted_iota(jnp.int32, (bn, bv), 1)
    w = jnp.where(col == tgt[:, None], w_hit, w_miss)
    wrow = jnp.sum(lp * w, axis=1)

    @pl.when((i == 0) & (j == 0))
    def _():
        out_ref[0] = 0.0
        out_ref[1] = 0.0

    out_ref[0] += jnp.sum(wrow * valid)

    @pl.when(j == 0)
    def _():
        out_ref[1] += jnp.sum(valid)


def kernel(log_probs, targets, triplets):
    n, v = log_probs.shape
    bn = 128
    bv = 32000
    nb = n // bn
    vb = v // bv

    tgt3 = targets.reshape(nb, 1, bn)

    sums = pl.pallas_call(
        functools.partial(_tc_body, bn=bn, bv=bv),
        grid=(nb, vb),
        in_specs=[
            pl.BlockSpec((1, 1, bn), lambda i, j: (i, 0, 0)),
            pl.BlockSpec((bn, bv), lambda i, j: (i, j)),
        ],
        out_specs=pl.BlockSpec(memory_space=pltpu.SMEM),
        out_shape=jax.ShapeDtypeStruct((2,), jnp.float32),
    )(tgt3, log_probs)

    return -sums[0] / jnp.maximum(sums[1], 1.0)
